# baseline (speedup 1.0000x reference)
import jax
import jax.numpy as jnp
from jax.experimental import pallas as pl
from jax.experimental.pallas import tpu as pltpu

_S = 64.0
_MARGIN = 0.5


def _body_factory(B, C, BR, BC):
    NC = pl.cdiv(C, BC)

    def body(x_ref, lab_ref, out_ref, m_ref, s_ref, p_ref):
        i = pl.program_id(0)
        j = pl.program_id(1)

        @pl.when(j == 0)
        def _init():
            m_ref[...] = jnp.full((BR, 1), -jnp.inf, jnp.float32)
            s_ref[...] = jnp.zeros((BR, 1), jnp.float32)
            p_ref[...] = jnp.zeros((BR, 1), jnp.float32)

        z = x_ref[...] * _S
        cols = j * BC + jax.lax.broadcasted_iota(jnp.int32, (BR, BC), 1)
        lab = lab_ref[...]
        hit = cols == lab
        z = jnp.where(hit, z - _S * _MARGIN, z)
        if C % BC != 0:
            z = jnp.where(cols < C, z, -jnp.inf)
        p_ref[...] += jnp.sum(jnp.where(hit, z, 0.0), axis=1, keepdims=True)
        m_old = m_ref[...]
        m_new = jnp.maximum(m_old, jnp.max(z, axis=1, keepdims=True))
        s_ref[...] = s_ref[...] * jnp.exp(m_old - m_new) + jnp.sum(
            jnp.exp(z - m_new), axis=1, keepdims=True
        )
        m_ref[...] = m_new

        @pl.when(j == NC - 1)
        def _finish():
            row_loss = m_ref[...] + jnp.log(s_ref[...]) - p_ref[...]
            tot = jnp.sum(row_loss) * (1.0 / B)

            @pl.when(i == 0)
            def _first():
                out_ref[...] = jnp.full((1, 1), tot, jnp.float32)

            @pl.when(i != 0)
            def _rest():
                out_ref[...] = out_ref[...] + tot

    return body, NC


def _grid_call(cosine, lab2d, BR, BC):
    B, C = cosine.shape
    body, NC = _body_factory(B, C, BR, BC)
    out = pl.pallas_call(
        body,
        grid=(B // BR, NC),
        in_specs=[
            pl.BlockSpec((BR, BC), lambda i, j: (i, j)),
            pl.BlockSpec((BR, 1), lambda i, j: (i, 0)),
        ],
        out_specs=pl.BlockSpec((1, 1), lambda i, j: (0, 0)),
        out_shape=jax.ShapeDtypeStruct((1, 1), jnp.float32),
        scratch_shapes=[
            pltpu.VMEM((BR, 1), jnp.float32),
            pltpu.VMEM((BR, 1), jnp.float32),
            pltpu.VMEM((BR, 1), jnp.float32),
        ],
        compiler_params=pltpu.CompilerParams(
            dimension_semantics=("arbitrary", "arbitrary")
        ),
    )(cosine, lab2d)
    return out[0, 0]


@jax.jit
def kernel(cosine, label):
    B, _ = cosine.shape
    lab2d = label.astype(jnp.int32).reshape(B, 1)
    return _grid_call(cosine, lab2d, 256, 2048)


# speedup vs baseline: 1.0280x; 1.0280x over previous
import jax
import jax.numpy as jnp
from jax.experimental import pallas as pl
from jax.experimental.pallas import tpu as pltpu

_S = 64.0
_MARGIN = 0.5
_C2 = _S * 1.4426950408889634


def _body_factory(B, C, BR, BC):
    NC = pl.cdiv(C, BC)

    def _accum(x, m_ref, s_ref):
        m_old = m_ref[...]
        m_new = jnp.maximum(m_old, jnp.max(x, axis=1, keepdims=True))
        mc = m_new * _C2
        s_ref[...] = s_ref[...] * jnp.exp2(m_old * _C2 - mc) + jnp.sum(
            jnp.exp2(x * _C2 - mc), axis=1, keepdims=True
        )
        m_ref[...] = m_new

    def body(x_ref, lab_ref, out_ref, m_ref, s_ref, p_ref):
        i = pl.program_id(0)
        j = pl.program_id(1)

        @pl.when(j == 0)
        def _init():
            m_ref[...] = jnp.full((BR, 1), -jnp.inf, jnp.float32)
            s_ref[...] = jnp.zeros((BR, 1), jnp.float32)
            p_ref[...] = jnp.zeros((BR, 1), jnp.float32)

        x = x_ref[...]
        cols = j * BC + jax.lax.broadcasted_iota(jnp.int32, (BR, BC), 1)
        hit = cols == lab_ref[...]
        p_ref[...] += jnp.sum(jnp.where(hit, x, 0.0), axis=1, keepdims=True)

        if C % BC != 0:
            @pl.when(j < NC - 1)
            def _fast():
                _accum(x, m_ref, s_ref)

            @pl.when(j == NC - 1)
            def _masked():
                _accum(jnp.where(cols < C, x, -jnp.inf), m_ref, s_ref)
        else:
            _accum(x, m_ref, s_ref)

        @pl.when(j == NC - 1)
        def _finish():
            m = m_ref[...]
            s = s_ref[...]
            a_x = p_ref[...]
            q = jnp.exp(-_S * _MARGIN)
            ea = jnp.exp2(a_x * _C2 - m * _C2)
            s_adj = jnp.maximum(s - ea * (1.0 - q), ea * q)
            row_loss = _S * m + jnp.log(s_adj) - _S * (a_x - _MARGIN)
            tot = jnp.sum(row_loss) * (1.0 / B)

            @pl.when(i == 0)
            def _first():
                out_ref[...] = jnp.full((1, 1), tot, jnp.float32)

            @pl.when(i != 0)
            def _rest():
                out_ref[...] = out_ref[...] + tot

    return body, NC


def _grid_call(cosine, lab2d, BR, BC):
    B, C = cosine.shape
    body, NC = _body_factory(B, C, BR, BC)
    out = pl.pallas_call(
        body,
        grid=(B // BR, NC),
        in_specs=[
            pl.BlockSpec((BR, BC), lambda i, j: (i, j)),
            pl.BlockSpec((BR, 1), lambda i, j: (i, 0)),
        ],
        out_specs=pl.BlockSpec((1, 1), lambda i, j: (0, 0)),
        out_shape=jax.ShapeDtypeStruct((1, 1), jnp.float32),
        scratch_shapes=[
            pltpu.VMEM((BR, 1), jnp.float32),
            pltpu.VMEM((BR, 1), jnp.float32),
            pltpu.VMEM((BR, 1), jnp.float32),
        ],
        compiler_params=pltpu.CompilerParams(
            dimension_semantics=("arbitrary", "arbitrary")
        ),
    )(cosine, lab2d)
    return out[0, 0]


@jax.jit
def kernel(cosine, label):
    B, _ = cosine.shape
    lab2d = label.astype(jnp.int32).reshape(B, 1)
    return _grid_call(cosine, lab2d, 256, 2048)


# speedup vs baseline: 1.2192x; 1.1860x over previous
import jax
import jax.numpy as jnp
from jax.experimental import pallas as pl
from jax.experimental.pallas import tpu as pltpu

_S = 64.0
_MARGIN = 0.5
_C2 = _S * 1.4426950408889634


def _body_factory(B, C, BR, BC):
    NC = pl.cdiv(C, BC)

    def _accum(x, m_ref, s_ref):
        m_old = m_ref[...]
        m_new = jnp.maximum(m_old, jnp.max(x, axis=1, keepdims=True))
        mc = m_new * _C2
        s_ref[...] = s_ref[...] * jnp.exp2(m_old * _C2 - mc) + jnp.sum(
            jnp.exp2(x * _C2 - mc), axis=1, keepdims=True
        )
        m_ref[...] = m_new

    def body(x_ref, lab_ref, out_ref, m_ref, s_ref, p_ref):
        i = pl.program_id(0)
        j = pl.program_id(1)

        @pl.when(j == 0)
        def _init():
            m_ref[...] = jnp.full((BR, 1), -jnp.inf, jnp.float32)
            s_ref[...] = jnp.zeros((BR, 1), jnp.float32)
            p_ref[...] = jnp.zeros((BR, 1), jnp.float32)

        x = x_ref[...]
        cols = j * BC + jax.lax.broadcasted_iota(jnp.int32, (BR, BC), 1)
        hit = cols == lab_ref[...]
        p_ref[...] += jnp.sum(jnp.where(hit, x, 0.0), axis=1, keepdims=True)

        if C % BC != 0:
            @pl.when(j < NC - 1)
            def _fast():
                _accum(x, m_ref, s_ref)

            @pl.when(j == NC - 1)
            def _masked():
                _accum(jnp.where(cols < C, x, -jnp.inf), m_ref, s_ref)
        else:
            _accum(x, m_ref, s_ref)

        @pl.when(j == NC - 1)
        def _finish():
            m = m_ref[...]
            s = s_ref[...]
            a_x = p_ref[...]
            q = jnp.exp(-_S * _MARGIN)
            ea = jnp.exp2(a_x * _C2 - m * _C2)
            s_adj = jnp.maximum(s - ea * (1.0 - q), ea * q)
            row_loss = _S * m + jnp.log(s_adj) - _S * (a_x - _MARGIN)
            tot = jnp.sum(row_loss) * (1.0 / B)

            @pl.when(i == 0)
            def _first():
                out_ref[...] = jnp.full((1, 1), tot, jnp.float32)

            @pl.when(i != 0)
            def _rest():
                out_ref[...] = out_ref[...] + tot

    return body, NC


def _grid_call(cosine, lab2d, BR, BC):
    B, C = cosine.shape
    body, NC = _body_factory(B, C, BR, BC)
    out = pl.pallas_call(
        body,
        grid=(B // BR, NC),
        in_specs=[
            pl.BlockSpec((BR, BC), lambda i, j: (i, j)),
            pl.BlockSpec((BR, 1), lambda i, j: (i, 0)),
        ],
        out_specs=pl.BlockSpec((1, 1), lambda i, j: (0, 0)),
        out_shape=jax.ShapeDtypeStruct((1, 1), jnp.float32),
        scratch_shapes=[
            pltpu.VMEM((BR, 1), jnp.float32),
            pltpu.VMEM((BR, 1), jnp.float32),
            pltpu.VMEM((BR, 1), jnp.float32),
        ],
        compiler_params=pltpu.CompilerParams(
            dimension_semantics=("arbitrary", "arbitrary")
        ),
    )(cosine, lab2d)
    return out[0, 0]


@jax.jit
def kernel(cosine, label):
    B, _ = cosine.shape
    lab2d = label.astype(jnp.int32).reshape(B, 1)
    return _grid_call(cosine, lab2d, 512, 4096)
